# baseline (speedup 1.0000x reference)
import functools

import jax
import jax.numpy as jnp
from jax import lax
from jax.experimental import pallas as pl
from jax.experimental.pallas import tpu as pltpu
from jax.experimental.pallas import tpu_sc as plsc

B = 32
A = 8732
C = 85
AB = 4368
NBLK = 2
APAD = NBLK * AB
NEG_FACTOR_K = 3
I32_MIN = -(2 ** 31)
I32_MAX = 2 ** 31 - 1


def _stage_a_body(pos_ref, pred_ref, gts_ref,
                  keys_ref, ce_ref, loc_ref, posce_ref, n_ref):
    b = pl.program_id(0)
    a = pl.program_id(1)
    x = pred_ref[0]
    g = gts_ref[0]
    p = pos_ref[0]

    xT = x.T
    gT = g.T

    sub = jax.lax.broadcasted_iota(jnp.int32, (C, AB), 0)
    lanev = jax.lax.broadcasted_iota(jnp.int32, (1, AB), 1)
    valid = (a * AB + lanev) < A
    posb = (p > 0) & valid

    d = xT[0:8, :] - gT[0:8, :]
    ad = jnp.abs(d)
    sl1 = jnp.where(sub[0:8, :] < 4,
                    jnp.where(ad < 1.0, 0.5 * d * d, ad - 0.5), 0.0)
    locv = jnp.sum(sl1, axis=0, keepdims=True)

    isc = sub >= 4
    xm = jnp.where(isc, xT, -jnp.inf)
    m = jnp.max(xm, axis=0, keepdims=True)
    e = jnp.exp(xm - m)
    s = jnp.sum(e, axis=0, keepdims=True)
    lse = jnp.log(s) + m
    bg = xT[C - 1:C, :]
    bl = jnp.maximum(lse - bg, 0.0)

    gbits = jax.lax.bitcast_convert_type(gT, jnp.int32)
    packed = jnp.where(isc, (gbits & ~127) | (127 - sub), -1)
    pmax = jnp.max(packed, axis=0, keepdims=True)
    lab = 127 - (pmax & 127)
    pa = jnp.sum(jnp.where(sub == lab, xT, 0.0), axis=0, keepdims=True)
    ce = lse - pa

    kbits = jax.lax.bitcast_convert_type(bl, jnp.int32)
    key = jnp.where(posb, I32_MIN, kbits)
    key = jnp.where(valid, key, -1)

    keys_ref[0] = key
    ce_ref[0] = ce

    @pl.when((b == 0) & (a == 0))
    def _():
        loc_ref[...] = jnp.zeros_like(loc_ref)
        posce_ref[...] = jnp.zeros_like(posce_ref)
        n_ref[...] = jnp.zeros_like(n_ref)

    loc_ref[...] += jnp.sum(jnp.where(posb, locv, 0.0))
    posce_ref[...] += jnp.sum(jnp.where(posb, ce, 0.0))
    n_ref[...] += jnp.sum(jnp.where(posb, 1.0, 0.0))


def _stage_b_body(keys_ref, ce_ref, neg_ref):
    keys = keys_ref[...]
    ce = ce_ref[...]

    posm = keys == I32_MIN
    pos_num = jnp.sum(posm.astype(jnp.int32), axis=1, keepdims=True)
    k = jnp.minimum(A - pos_num, NEG_FACTOR_K * pos_num)

    lo = jnp.zeros_like(k)
    hi = jnp.full_like(k, I32_MAX - 1)

    def body(_, lohi):
        lo, hi = lohi
        mid = lo + ((hi - lo + 1) >> 1)
        cnt = jnp.sum((keys >= mid).astype(jnp.int32), axis=1, keepdims=True)
        ge = cnt >= k
        return (jnp.where(ge, mid, lo), jnp.where(ge, hi, mid - 1))

    lo, _ = jax.lax.fori_loop(0, 31, body, (lo, hi))
    vstar = lo

    gt = keys > vstar
    c_gt = jnp.sum(gt.astype(jnp.int32), axis=1, keepdims=True)
    need = k - c_gt
    eq = keys == vstar
    lane = jax.lax.broadcasted_iota(jnp.int32, (B, APAD), 1)

    lo2 = jnp.zeros_like(k)
    hi2 = jnp.full_like(k, APAD)

    def body2(_, lohi):
        lo2, hi2 = lohi
        mid = (lo2 + hi2) >> 1
        gcnt = jnp.sum((eq & (lane < mid)).astype(jnp.int32),
                       axis=1, keepdims=True)
        geq = gcnt >= need
        return (jnp.where(geq, lo2, mid + 1), jnp.where(geq, mid, hi2))

    tstar, _ = jax.lax.fori_loop(0, 14, body2, (lo2, hi2))

    sel = gt | (eq & (lane < tstar))
    neg_ref[...] = jnp.zeros_like(neg_ref) + jnp.sum(jnp.where(sel, ce, 0.0))


NCH = APAD // 16
_RADIX_DIGITS = ((23, 8), (15, 8), (7, 8), (0, 7))


_GDN = lax.GatherDimensionNumbers(
    offset_dims=(), collapsed_slice_dims=(0,), start_index_map=(0,))


def _splat_last(x):
    idx = jnp.full((16, 1), 15, jnp.int32)
    return lax.gather(x, idx, dimension_numbers=_GDN, slice_sizes=(1,),
                      mode=lax.GatherScatterMode.PROMISE_IN_BOUNDS)


def _sc_topk_body(keys_hbm, ce_hbm, out_hbm, keys_v, ce_v, hist, out_v):
    wid = lax.axis_index("s") * 2 + lax.axis_index("c")
    pltpu.sync_copy(keys_hbm.at[wid], keys_v)
    pltpu.sync_copy(ce_hbm.at[wid], ce_v)

    lane = lax.iota(jnp.int32, 16)
    z16i = jnp.zeros((16,), jnp.int32)
    ones16 = jnp.ones((16,), jnp.int32)
    laneh = lane * 256

    def pb(i, acc):
        kv = keys_v[pl.ds(i * 16, 16)]
        return acc + plsc.all_reduce_population_count(kv == I32_MIN)

    pos_num = lax.fori_loop(0, NCH, pb, z16i)
    k = jnp.minimum(A - pos_num, NEG_FACTOR_K * pos_num)

    pref = z16i
    r = k
    for sh, w in _RADIX_DIGITS:
        top = sh + w
        maskw = (1 << w) - 1

        def zb(i, c):
            hist[pl.ds(i * 16, 16)] = z16i
            return c

        lax.fori_loop(0, (16 * 256) // 16, zb, 0)

        prefsh = lax.shift_right_arithmetic(pref, top)

        def hb(i, c):
            kv = keys_v[pl.ds(i * 16, 16)]
            mv = lax.shift_right_arithmetic(kv, top) == prefsh
            bucket = jnp.bitwise_and(lax.shift_right_arithmetic(kv, sh), maskw)
            plsc.addupdate_scatter(hist, [laneh + bucket], ones16, mask=mv)
            return c

        lax.fori_loop(0, NCH, hb, 0)

        def sb(j, carry):
            run, cmax, smax = carry
            c = 15 - j
            t = z16i
            for l in range(16):
                t = t + hist[pl.ds(l * 256 + c * 16, 16)]
            ps = plsc.cumsum(t)
            csum = _splat_last(ps)
            sfxv = (run + csum - ps) + t
            ids = c * 16 + lane
            m = sfxv >= r
            idsh = lax.shift_left(ids, 14)
            cmax = jnp.maximum(cmax, jnp.where(m, idsh | t, -1))
            smax = jnp.maximum(smax, jnp.where(m, idsh | sfxv, -1))
            return (run + csum, cmax, smax)

        neg1 = jnp.full((16,), -1, jnp.int32)
        _, cmaxv, smaxv = lax.fori_loop(0, 16, sb, (z16i, neg1, neg1))
        cpack = _splat_last(plsc.cummax(cmaxv))
        spack = _splat_last(plsc.cummax(smaxv))
        bstar = lax.shift_right_logical(cpack, 14)
        cnt_b = jnp.bitwise_and(cpack, 0x3FFF)
        sfx_b = jnp.bitwise_and(spack, 0x3FFF)
        r = r - (sfx_b - cnt_b)
        pref = jnp.bitwise_or(pref, lax.shift_left(bstar, sh))

    vstar = pref
    need = r

    def fb(i, carry):
        acc, cnt_eq = carry
        kv = keys_v[pl.ds(i * 16, 16)]
        cv = ce_v[pl.ds(i * 16, 16)]
        acc = acc + jnp.where(kv > vstar, cv, 0.0)
        eqm = kv == vstar
        eqi = jnp.where(eqm, ones16, z16i)
        pe = plsc.cumsum(eqi)
        excl = cnt_eq + pe - eqi
        sel = eqm & (excl < need)
        acc = acc + jnp.where(sel, cv, 0.0)
        cnt_eq = cnt_eq + plsc.all_reduce_population_count(eqm)
        return (acc, cnt_eq)

    accv, _ = lax.fori_loop(
        0, NCH, fb, (jnp.zeros((16,), jnp.float32), z16i))

    out_v[...] = accv
    pltpu.sync_copy(out_v, out_hbm.at[wid])


@functools.partial(
    pl.kernel,
    mesh=plsc.VectorSubcoreMesh(core_axis_name="c", subcore_axis_name="s"),
    compiler_params=pltpu.CompilerParams(needs_layout_passes=False),
    out_type=jax.ShapeDtypeStruct((B, 16), jnp.float32),
    scratch_types=[
        pltpu.VMEM((APAD,), jnp.int32),
        pltpu.VMEM((APAD,), jnp.float32),
        pltpu.VMEM((16 * 256,), jnp.int32),
        pltpu.VMEM((16,), jnp.float32),
    ],
)
def _sc_topk(keys_hbm, ce_hbm, out_hbm, keys_v, ce_v, hist, out_v):
    _sc_topk_body(keys_hbm, ce_hbm, out_hbm, keys_v, ce_v, hist, out_v)


def kernel(pos_indicator, predicts, gts):
    pos = pos_indicator.astype(jnp.int32)
    pos = jnp.pad(pos, ((0, 0), (0, APAD - A)))
    posr = pos.reshape(B * NBLK, 1, AB)

    keys, cev, locsum, posce, n = pl.pallas_call(
        _stage_a_body,
        grid=(B, NBLK),
        in_specs=[
            pl.BlockSpec((1, 1, AB), lambda b, a: (b * NBLK + a, 0, 0)),
            pl.BlockSpec((1, AB, C), lambda b, a: (b, a, 0)),
            pl.BlockSpec((1, AB, C), lambda b, a: (b, a, 0)),
        ],
        out_specs=[
            pl.BlockSpec((1, 1, AB), lambda b, a: (b * NBLK + a, 0, 0)),
            pl.BlockSpec((1, 1, AB), lambda b, a: (b * NBLK + a, 0, 0)),
            pl.BlockSpec((1, 1), lambda b, a: (0, 0)),
            pl.BlockSpec((1, 1), lambda b, a: (0, 0)),
            pl.BlockSpec((1, 1), lambda b, a: (0, 0)),
        ],
        out_shape=[
            jax.ShapeDtypeStruct((B * NBLK, 1, AB), jnp.int32),
            jax.ShapeDtypeStruct((B * NBLK, 1, AB), jnp.float32),
            jax.ShapeDtypeStruct((1, 1), jnp.float32),
            jax.ShapeDtypeStruct((1, 1), jnp.float32),
            jax.ShapeDtypeStruct((1, 1), jnp.float32),
        ],
    )(posr, predicts, gts)

    keys2 = keys.reshape(B, APAD)
    ce2 = cev.reshape(B, APAD)

    negrows = _sc_topk(keys2, ce2)
    negsum = jnp.sum(negrows)

    nn = n[0, 0]
    conf_loss = (posce[0, 0] + negsum) / nn
    loc_loss = locsum[0, 0] / nn
    return (conf_loss, loc_loss)


# speedup vs baseline: 1.0681x; 1.0681x over previous
import functools

import jax
import jax.numpy as jnp
from jax import lax
from jax.experimental import pallas as pl
from jax.experimental.pallas import tpu as pltpu
from jax.experimental.pallas import tpu_sc as plsc

B = 32
A = 8732
C = 85
AB = 8736
NBLK = 1
APAD = NBLK * AB
NEG_FACTOR_K = 3
I32_MIN = -(2 ** 31)
I32_MAX = 2 ** 31 - 1


def _stage_a_body(pos_ref, pred_ref, gts_ref,
                  keys_ref, ce_ref, loc_ref, posce_ref, n_ref):
    b = pl.program_id(0)
    a = pl.program_id(1)
    x = pred_ref[0]
    g = gts_ref[0]
    p = pos_ref[0]

    xT = x.T
    gT = g.T

    sub = jax.lax.broadcasted_iota(jnp.int32, (C, AB), 0)
    lanev = jax.lax.broadcasted_iota(jnp.int32, (1, AB), 1)
    valid = (a * AB + lanev) < A
    posb = (p > 0) & valid

    d = xT[0:8, :] - gT[0:8, :]
    ad = jnp.abs(d)
    sl1 = jnp.where(sub[0:8, :] < 4,
                    jnp.where(ad < 1.0, 0.5 * d * d, ad - 0.5), 0.0)
    locv = jnp.sum(sl1, axis=0, keepdims=True)

    isc = sub >= 4
    xm = jnp.where(isc, xT, -jnp.inf)
    m = jnp.max(xm, axis=0, keepdims=True)
    e = jnp.exp(xm - m)
    s = jnp.sum(e, axis=0, keepdims=True)
    lse = jnp.log(s) + m
    bg = xT[C - 1:C, :]
    bl = jnp.maximum(lse - bg, 0.0)

    gbits = jax.lax.bitcast_convert_type(gT, jnp.int32)
    packed = jnp.where(isc, (gbits & ~127) | (127 - sub), -1)
    pmax = jnp.max(packed, axis=0, keepdims=True)
    lab = 127 - (pmax & 127)
    pa = jnp.sum(jnp.where(sub == lab, xT, 0.0), axis=0, keepdims=True)
    ce = lse - pa

    kbits = jax.lax.bitcast_convert_type(bl, jnp.int32)
    key = jnp.where(posb, I32_MIN, kbits)
    key = jnp.where(valid, key, -1)

    keys_ref[0] = key
    ce_ref[0] = ce

    @pl.when((b == 0) & (a == 0))
    def _():
        loc_ref[...] = jnp.zeros_like(loc_ref)
        posce_ref[...] = jnp.zeros_like(posce_ref)
        n_ref[...] = jnp.zeros_like(n_ref)

    loc_ref[...] += jnp.sum(jnp.where(posb, locv, 0.0))
    posce_ref[...] += jnp.sum(jnp.where(posb, ce, 0.0))
    n_ref[...] += jnp.sum(jnp.where(posb, 1.0, 0.0))


def _stage_b_body(keys_ref, ce_ref, neg_ref):
    keys = keys_ref[...]
    ce = ce_ref[...]

    posm = keys == I32_MIN
    pos_num = jnp.sum(posm.astype(jnp.int32), axis=1, keepdims=True)
    k = jnp.minimum(A - pos_num, NEG_FACTOR_K * pos_num)

    lo = jnp.zeros_like(k)
    hi = jnp.full_like(k, I32_MAX - 1)

    def body(_, lohi):
        lo, hi = lohi
        mid = lo + ((hi - lo + 1) >> 1)
        cnt = jnp.sum((keys >= mid).astype(jnp.int32), axis=1, keepdims=True)
        ge = cnt >= k
        return (jnp.where(ge, mid, lo), jnp.where(ge, hi, mid - 1))

    lo, _ = jax.lax.fori_loop(0, 31, body, (lo, hi))
    vstar = lo

    gt = keys > vstar
    c_gt = jnp.sum(gt.astype(jnp.int32), axis=1, keepdims=True)
    need = k - c_gt
    eq = keys == vstar
    lane = jax.lax.broadcasted_iota(jnp.int32, (B, APAD), 1)

    lo2 = jnp.zeros_like(k)
    hi2 = jnp.full_like(k, APAD)

    def body2(_, lohi):
        lo2, hi2 = lohi
        mid = (lo2 + hi2) >> 1
        gcnt = jnp.sum((eq & (lane < mid)).astype(jnp.int32),
                       axis=1, keepdims=True)
        geq = gcnt >= need
        return (jnp.where(geq, lo2, mid + 1), jnp.where(geq, mid, hi2))

    tstar, _ = jax.lax.fori_loop(0, 14, body2, (lo2, hi2))

    sel = gt | (eq & (lane < tstar))
    neg_ref[...] = jnp.zeros_like(neg_ref) + jnp.sum(jnp.where(sel, ce, 0.0))


NCH = APAD // 16
_RADIX_DIGITS = ((23, 8), (15, 8), (7, 8), (0, 7))


_GDN = lax.GatherDimensionNumbers(
    offset_dims=(), collapsed_slice_dims=(0,), start_index_map=(0,))


def _splat_last(x):
    idx = jnp.full((16, 1), 15, jnp.int32)
    return lax.gather(x, idx, dimension_numbers=_GDN, slice_sizes=(1,),
                      mode=lax.GatherScatterMode.PROMISE_IN_BOUNDS)


def _sc_topk_body(keys_hbm, ce_hbm, out_hbm, keys_v, ce_v, hist, out_v):
    wid = lax.axis_index("s") * 2 + lax.axis_index("c")
    pltpu.sync_copy(keys_hbm.at[wid], keys_v)
    pltpu.sync_copy(ce_hbm.at[wid], ce_v)

    lane = lax.iota(jnp.int32, 16)
    z16i = jnp.zeros((16,), jnp.int32)
    ones16 = jnp.ones((16,), jnp.int32)
    laneh = lane * 256

    pref = z16i
    r = None
    k = None
    for sh, w in _RADIX_DIGITS:
        top = sh + w
        maskw = (1 << w) - 1

        def zb(i, c):
            hist[pl.ds(i * 16, 16)] = z16i
            return c

        lax.fori_loop(0, (16 * 256) // 16, zb, 0)

        prefsh = lax.shift_right_arithmetic(pref, top)

        if r is None:
            def hb1(i, acc):
                kv = keys_v[pl.ds(i * 16, 16)]
                mv = lax.shift_right_arithmetic(kv, top) == prefsh
                bucket = jnp.bitwise_and(
                    lax.shift_right_arithmetic(kv, sh), maskw)
                plsc.addupdate_scatter(hist, [laneh + bucket], ones16, mask=mv)
                return acc + plsc.all_reduce_population_count(kv == I32_MIN)

            pos_num = lax.fori_loop(0, NCH, hb1, z16i)
            k = jnp.minimum(A - pos_num, NEG_FACTOR_K * pos_num)
            r = k
        else:
            def hb(i, c):
                kv = keys_v[pl.ds(i * 16, 16)]
                mv = lax.shift_right_arithmetic(kv, top) == prefsh
                bucket = jnp.bitwise_and(
                    lax.shift_right_arithmetic(kv, sh), maskw)
                plsc.addupdate_scatter(hist, [laneh + bucket], ones16, mask=mv)
                return c

            lax.fori_loop(0, NCH, hb, 0)

        def sb(j, carry):
            run, cmax, smax = carry
            c = 15 - j
            t = z16i
            for l in range(16):
                t = t + hist[pl.ds(l * 256 + c * 16, 16)]
            ps = plsc.cumsum(t)
            csum = _splat_last(ps)
            sfxv = (run + csum - ps) + t
            ids = c * 16 + lane
            m = sfxv >= r
            idsh = lax.shift_left(ids, 14)
            cmax = jnp.maximum(cmax, jnp.where(m, idsh | t, -1))
            smax = jnp.maximum(smax, jnp.where(m, idsh | sfxv, -1))
            return (run + csum, cmax, smax)

        neg1 = jnp.full((16,), -1, jnp.int32)
        _, cmaxv, smaxv = lax.fori_loop(0, 16, sb, (z16i, neg1, neg1))
        cpack = _splat_last(plsc.cummax(cmaxv))
        spack = _splat_last(plsc.cummax(smaxv))
        bstar = lax.shift_right_logical(cpack, 14)
        cnt_b = jnp.bitwise_and(cpack, 0x3FFF)
        sfx_b = jnp.bitwise_and(spack, 0x3FFF)
        r = r - (sfx_b - cnt_b)
        pref = jnp.bitwise_or(pref, lax.shift_left(bstar, sh))

    vstar = pref
    need = r

    def fb(i, carry):
        acc, cnt_eq = carry
        kv = keys_v[pl.ds(i * 16, 16)]
        cv = ce_v[pl.ds(i * 16, 16)]
        acc = acc + jnp.where(kv > vstar, cv, 0.0)
        eqm = kv == vstar
        eqi = jnp.where(eqm, ones16, z16i)
        pe = plsc.cumsum(eqi)
        excl = cnt_eq + pe - eqi
        sel = eqm & (excl < need)
        acc = acc + jnp.where(sel, cv, 0.0)
        cnt_eq = cnt_eq + plsc.all_reduce_population_count(eqm)
        return (acc, cnt_eq)

    accv, _ = lax.fori_loop(
        0, NCH, fb, (jnp.zeros((16,), jnp.float32), z16i))

    out_v[...] = accv
    pltpu.sync_copy(out_v, out_hbm.at[wid])


@functools.partial(
    pl.kernel,
    mesh=plsc.VectorSubcoreMesh(core_axis_name="c", subcore_axis_name="s"),
    compiler_params=pltpu.CompilerParams(needs_layout_passes=False),
    out_type=jax.ShapeDtypeStruct((B, 16), jnp.float32),
    scratch_types=[
        pltpu.VMEM((APAD,), jnp.int32),
        pltpu.VMEM((APAD,), jnp.float32),
        pltpu.VMEM((16 * 256,), jnp.int32),
        pltpu.VMEM((16,), jnp.float32),
    ],
)
def _sc_topk(keys_hbm, ce_hbm, out_hbm, keys_v, ce_v, hist, out_v):
    _sc_topk_body(keys_hbm, ce_hbm, out_hbm, keys_v, ce_v, hist, out_v)


def kernel(pos_indicator, predicts, gts):
    pos = pos_indicator.astype(jnp.int32)
    pos = jnp.pad(pos, ((0, 0), (0, APAD - A)))
    posr = pos.reshape(B * NBLK, 1, AB)

    keys, cev, locsum, posce, n = pl.pallas_call(
        _stage_a_body,
        grid=(B, NBLK),
        in_specs=[
            pl.BlockSpec((1, 1, AB), lambda b, a: (b * NBLK + a, 0, 0)),
            pl.BlockSpec((1, AB, C), lambda b, a: (b, a, 0)),
            pl.BlockSpec((1, AB, C), lambda b, a: (b, a, 0)),
        ],
        out_specs=[
            pl.BlockSpec((1, 1, AB), lambda b, a: (b * NBLK + a, 0, 0)),
            pl.BlockSpec((1, 1, AB), lambda b, a: (b * NBLK + a, 0, 0)),
            pl.BlockSpec((1, 1), lambda b, a: (0, 0)),
            pl.BlockSpec((1, 1), lambda b, a: (0, 0)),
            pl.BlockSpec((1, 1), lambda b, a: (0, 0)),
        ],
        out_shape=[
            jax.ShapeDtypeStruct((B * NBLK, 1, AB), jnp.int32),
            jax.ShapeDtypeStruct((B * NBLK, 1, AB), jnp.float32),
            jax.ShapeDtypeStruct((1, 1), jnp.float32),
            jax.ShapeDtypeStruct((1, 1), jnp.float32),
            jax.ShapeDtypeStruct((1, 1), jnp.float32),
        ],
    )(posr, predicts, gts)

    keys2 = keys.reshape(B, APAD)
    ce2 = cev.reshape(B, APAD)

    negrows = _sc_topk(keys2, ce2)
    negsum = jnp.sum(negrows)

    nn = n[0, 0]
    conf_loss = (posce[0, 0] + negsum) / nn
    loc_loss = locsum[0, 0] / nn
    return (conf_loss, loc_loss)
